# Initial kernel scaffold; baseline (speedup 1.0000x reference)
import jax
import jax.numpy as jnp
from jax.experimental import pallas as pl


def kernel(x, t, emb, W, b):
    raise NotImplementedError("write your pallas kernel here")


# speedup vs baseline: 2.1812x; 2.1812x over previous
import functools

import jax
import jax.numpy as jnp
from jax import lax
from jax.experimental import pallas as pl
from jax.experimental.pallas import tpu as pltpu
from jax.experimental.pallas import tpu_sc as plsc

N = 320000
D = 128
V = 1024


def _proj_body(emb_ref, w_ref, b_ref, out_ref):
    out_ref[:] = lax.dot_general(
        emb_ref[:], w_ref[:],
        dimension_numbers=(((1,), (1,)), ((), ())),
        preferred_element_type=jnp.float32,
    ) + b_ref[:]


def _project(emb, W, b):
    return pl.pallas_call(
        _proj_body,
        out_shape=jax.ShapeDtypeStruct((V, D), jnp.float32),
    )(emb, W, b.reshape(1, D))


_info = plsc.get_sparse_core_info()
_NC, _NS = _info.num_cores, _info.num_subcores
_NW = _NC * _NS
CHUNK = 128
NCHUNKS = N // CHUNK
_FULL = NCHUNKS // _NW
_REM = NCHUNKS % _NW

_mesh = plsc.VectorSubcoreMesh(core_axis_name="c", subcore_axis_name="s")


@functools.partial(
    pl.kernel,
    mesh=_mesh,
    out_type=jax.ShapeDtypeStruct((N, D), jnp.float32),
    scratch_types=[
        pltpu.VMEM((CHUNK,), jnp.int32),
        pltpu.VMEM((CHUNK, D), jnp.float32),
        pltpu.VMEM((CHUNK, D), jnp.float32),
        pltpu.SemaphoreType.DMA,
    ],
)
def _sc_add(p_hbm, x_hbm, t_hbm, out_hbm, idx_v, g_v, x_v, sem):
    wid = lax.axis_index("s") * _NC + lax.axis_index("c")

    def do_chunk(ci):
        off = ci * CHUNK
        pltpu.sync_copy(t_hbm.at[pl.ds(off, CHUNK)], idx_v)
        gcp = pltpu.async_copy(p_hbm.at[idx_v], g_v, sem)
        pltpu.sync_copy(x_hbm.at[pl.ds(off, CHUNK), :], x_v)
        gcp.wait()

        def add_row(r, carry):
            for j in range(D // 16):
                sl = pl.ds(j * 16, 16)
                g_v[r, sl] = g_v[r, sl] + x_v[r, sl]
            return carry

        lax.fori_loop(0, CHUNK, add_row, 0)
        pltpu.sync_copy(g_v, out_hbm.at[pl.ds(off, CHUNK), :])

    def loop_body(k, carry):
        do_chunk(wid + k * _NW)
        return carry

    lax.fori_loop(0, _FULL, loop_body, 0)

    @pl.when(wid < _REM)
    def _():
        do_chunk(_FULL * _NW + wid)


def kernel(x, t, emb, W, b):
    P = _project(emb, W, b)
    return _sc_add(P, x, t)


# speedup vs baseline: 2.8451x; 1.3044x over previous
import functools

import jax
import jax.numpy as jnp
from jax import lax
from jax.experimental import pallas as pl
from jax.experimental.pallas import tpu as pltpu
from jax.experimental.pallas import tpu_sc as plsc

N = 320000
D = 128
V = 1024


def _proj_body(emb_ref, w_ref, b_ref, out_ref):
    out_ref[:] = lax.dot_general(
        emb_ref[:], w_ref[:],
        dimension_numbers=(((1,), (1,)), ((), ())),
        preferred_element_type=jnp.float32,
    ) + b_ref[:]


def _project(emb, W, b):
    return pl.pallas_call(
        _proj_body,
        out_shape=jax.ShapeDtypeStruct((V, D), jnp.float32),
    )(emb, W, b.reshape(1, D))


_info = plsc.get_sparse_core_info()
_NC, _NS = _info.num_cores, _info.num_subcores
_NW = _NC * _NS
ROWS_W = N // _NW
CF = 128
NF = ROWS_W // CF
TAIL = ROWS_W - NF * CF

_mesh = plsc.VectorSubcoreMesh(core_axis_name="c", subcore_axis_name="s")


@functools.partial(
    pl.kernel,
    mesh=_mesh,
    out_type=jax.ShapeDtypeStruct((N, D), jnp.float32),
    scratch_types=[
        pltpu.VMEM((CF,), jnp.int32),
        pltpu.VMEM((CF,), jnp.int32),
        pltpu.VMEM((CF, D), jnp.float32),
        pltpu.VMEM((CF, D), jnp.float32),
        pltpu.VMEM((CF, D), jnp.float32),
        pltpu.VMEM((CF, D), jnp.float32),
        pltpu.VMEM((TAIL,), jnp.int32),
        pltpu.VMEM((TAIL, D), jnp.float32),
        pltpu.VMEM((TAIL, D), jnp.float32),
        pltpu.SemaphoreType.DMA,
        pltpu.SemaphoreType.DMA,
        pltpu.SemaphoreType.DMA,
        pltpu.SemaphoreType.DMA,
        pltpu.SemaphoreType.DMA,
        pltpu.SemaphoreType.DMA,
        pltpu.SemaphoreType.DMA,
    ],
)
def _sc_add(p_hbm, x_hbm, t_hbm, out_hbm,
            i0, i1, g0, g1, x0, x1, it, gt, xt,
            si0, si1, sx0, sx1, so0, so1, sgath):
    idx = (i0, i1)
    g = (g0, g1)
    x = (x0, x1)
    si = (si0, si1)
    sx = (sx0, sx1)
    so = (so0, so1)

    wid = lax.axis_index("s") * _NC + lax.axis_index("c")
    base = wid * ROWS_W

    def fire_in(c, b):
        pltpu.async_copy(t_hbm.at[pl.ds(base + c * CF, CF)], idx[b], si[b])
        pltpu.async_copy(x_hbm.at[pl.ds(base + c * CF, CF), :], x[b], sx[b])

    def wait_idx(c, b):
        pltpu.make_async_copy(
            t_hbm.at[pl.ds(base + c * CF, CF)], idx[b], si[b]).wait()

    def wait_x(c, b):
        pltpu.make_async_copy(
            x_hbm.at[pl.ds(base + c * CF, CF), :], x[b], sx[b]).wait()

    def wait_out(c, b):
        pltpu.make_async_copy(
            g[b], out_hbm.at[pl.ds(base + c * CF, CF), :], so[b]).wait()

    def accumulate(gb, xb, rows):
        def body(r4, carry):
            r = r4 * 4
            for rr in range(4):
                for j in range(D // 16):
                    sl = pl.ds(j * 16, 16)
                    plsc.addupdate(gb.at[r + rr, sl], xb[r + rr, sl])
            return carry
        lax.fori_loop(0, rows // 4, body, 0)

    fire_in(0, 0)

    def step(k, carry):
        for b in range(2):
            c = k * 2 + b
            bn = 1 - b

            @pl.when(c >= 2)
            def _():
                wait_out(c - 2, b)

            wait_idx(c, b)
            gcp = pltpu.async_copy(p_hbm.at[idx[b]], g[b], sgath)

            @pl.when(c + 1 < NF)
            def _():
                fire_in(c + 1, bn)

            gcp.wait()
            wait_x(c, b)
            accumulate(g[b], x[b], CF)
            pltpu.async_copy(
                g[b], out_hbm.at[pl.ds(base + c * CF, CF), :], so[b])
        return carry

    lax.fori_loop(0, NF // 2, step, 0)

    wait_out(NF - 2, 0)
    wait_out(NF - 1, 1)

    toff = base + NF * CF
    pltpu.sync_copy(t_hbm.at[pl.ds(toff, TAIL)], it)
    pltpu.async_copy(p_hbm.at[it], gt, sgath).wait()
    pltpu.sync_copy(x_hbm.at[pl.ds(toff, TAIL), :], xt)
    for r in range(TAIL):
        for j in range(D // 16):
            sl = pl.ds(j * 16, 16)
            plsc.addupdate(gt.at[r, sl], xt[r, sl])
    pltpu.sync_copy(gt, out_hbm.at[pl.ds(toff, TAIL), :])


def kernel(x, t, emb, W, b):
    P = _project(emb, W, b)
    return _sc_add(P, x, t)
